# baseline (speedup 1.0000x reference)
import functools

import jax
import jax.numpy as jnp
from jax import lax
from jax.experimental import pallas as pl
from jax.experimental.pallas import tpu as pltpu
from jax.experimental.pallas import tpu_sc as plsc

VOCAB = 1000000
DIM = 64
B_TOK = 16384
SEQ = 50

_info = plsc.get_sparse_core_info()
NC, NS, NL = _info.num_cores, _info.num_subcores, _info.num_lanes
NW = NC * NS

BLK = 128
NBH = B_TOK // BLK
UNITS = SEQ * NBH
UNITS_PER_W = UNITS // NW
PAD = BLK + 1


def _make_gather():
  mesh = plsc.VectorSubcoreMesh(core_axis_name="c", subcore_axis_name="s")

  @functools.partial(
      pl.kernel,
      mesh=mesh,
      compiler_params=pltpu.CompilerParams(
          use_tc_tiling_on_sc=False, needs_layout_passes=False),
      out_type=jax.ShapeDtypeStruct((SEQ, 8, NBH, 8, BLK), jnp.float32),
      scratch_types=[
          pltpu.VMEM((2, BLK), jnp.int32),
          pltpu.VMEM((2, BLK, DIM), jnp.float32),
          pltpu.VMEM((2, DIM, PAD), jnp.float32),
          pltpu.SemaphoreType.DMA,
          pltpu.SemaphoreType.DMA,
          pltpu.SemaphoreType.DMA,
          pltpu.SemaphoreType.DMA,
          pltpu.SemaphoreType.DMA,
          pltpu.SemaphoreType.DMA,
      ],
  )
  def gather_kernel(xT_hbm, table_hbm, out_hbm, idx_v, rows_v, tr_v,
                    isem0, isem1, gsem0, gsem1, wsem0, wsem1):
    wid = lax.axis_index("s") * NC + lax.axis_index("c")
    base_u = wid * UNITS_PER_W
    isem = (isem0, isem1)
    gsem = (gsem0, gsem1)
    wsem = (wsem0, wsem1)
    lanes = lax.iota(jnp.int32, NL)

    def unit_sb(u):
      gu = base_u + u
      return gu // NBH, gu % NBH

    def idx_src(u):
      s, bh = unit_sb(u)
      return xT_hbm.at[s, pl.ds(bh * BLK, BLK)]

    def issue_gather(slot):
      pltpu.async_copy(table_hbm.at[idx_v.at[slot]], rows_v.at[slot],
                       gsem[slot])

    def wait_gather(slot):
      pltpu.make_async_copy(
          table_hbm.at[pl.ds(0, BLK)], rows_v.at[slot], gsem[slot]).wait()

    def wait_idx(slot):
      pltpu.make_async_copy(idx_src(0), idx_v.at[slot], isem[slot]).wait()

    def wait_wb(slot):
      for _ in range(8):
        pltpu.make_async_copy(
            tr_v.at[slot, pl.ds(0, 8), pl.ds(0, BLK)], out_hbm.at[0, 0, 0],
            wsem[slot]).wait()

    dvecs = [lanes + k * NL for k in range(DIM // NL)]

    def transpose(slot):
      @plsc.parallel_loop(0, BLK, unroll=8)
      def _(t):
        tvec = lanes * 0 + t
        for k in range(DIM // NL):
          vals = plsc.load_gather(rows_v.at[slot, t], [dvecs[k]])
          plsc.store_scatter(tr_v.at[slot], [dvecs[k], tvec], vals)

    def writeback(u, slot):
      s, bh = unit_sb(u)
      for dh in range(8):
        pltpu.async_copy(
            tr_v.at[slot, pl.ds(8 * dh, 8), pl.ds(0, BLK)],
            out_hbm.at[s, dh, bh], wsem[slot])

    pltpu.sync_copy(idx_src(0), idx_v.at[0])
    issue_gather(0)
    pltpu.async_copy(idx_src(1), idx_v.at[1], isem[1])

    def unit_body(u, cur, nxt):
      @pl.when(u + 1 < UNITS_PER_W)
      def _():
        wait_idx(nxt)
        issue_gather(nxt)
      wait_gather(cur)
      @pl.when(u + 2 < UNITS_PER_W)
      def _():
        pltpu.async_copy(idx_src(u + 2), idx_v.at[cur], isem[cur])
      @pl.when(u >= 2)
      def _():
        wait_wb(cur)
      transpose(cur)
      writeback(u, cur)

    @pl.loop(0, UNITS_PER_W // 2)
    def _(i):
      unit_body(2 * i, 0, 1)
      unit_body(2 * i + 1, 1, 0)

    wait_wb(0)
    wait_wb(1)

  return gather_kernel


_gather = _make_gather()


def kernel(x, weight):
  xT = x.T
  out5 = _gather(xT, weight)
  return out5.transpose(2, 4, 0, 1, 3).reshape(B_TOK, SEQ, DIM)
